# baseline (speedup 1.0000x reference)
import jax
import jax.numpy as jnp
from jax.experimental import pallas as pl
from jax.experimental.pallas import tpu as pltpu

_DEPTH = 1000
_ROWS_PER_BLOCK = 512


def _one_hot_block(idx_ref, val_ref, out_ref):
    idx = idx_ref[...]
    iota = jax.lax.broadcasted_iota(jnp.int32, (idx.shape[0], _DEPTH), 1)
    off = val_ref[0]
    on = val_ref[1]
    out_ref[...] = jnp.where(iota == idx, on, off)


def kernel(indices, values):
    n = indices.size
    idx_flat = indices.reshape(n, 1)
    grid = n // _ROWS_PER_BLOCK
    out = pl.pallas_call(
        _one_hot_block,
        grid=(grid,),
        in_specs=[
            pl.BlockSpec((_ROWS_PER_BLOCK, 1), lambda i: (i, 0)),
            pl.BlockSpec(memory_space=pltpu.SMEM),
        ],
        out_specs=pl.BlockSpec((_ROWS_PER_BLOCK, _DEPTH), lambda i: (i, 0)),
        out_shape=jax.ShapeDtypeStruct((n, _DEPTH), jnp.float32),
    )(idx_flat, values)
    return out.reshape(*indices.shape, _DEPTH)
